# baseline (speedup 1.0000x reference)
import functools

import jax
import jax.numpy as jnp
from jax import lax
from jax.experimental import pallas as pl
from jax.experimental.pallas import tpu as pltpu
from jax.experimental.pallas import tpu_sc as plsc

N_NODES = 10000
IN_CH = 128
OUT_CH = 128
N_EDGES = 320000

NC = 2
NS = 16
NW = NC * NS
EPW = N_EDGES // NW
CHUNK = 125
CPAD = 128
NCHUNK = EPW // CHUNK
NV = IN_CH // 16
ACC_N = N_NODES + 8
ZSTRIPE = 125
NSTRIPE = N_NODES // ZSTRIPE
PERIOD = 6


def _final_body(x_ref, z_ref, pe_ref, wt_ref, wb_ref, b_ref, o_ref):
    z = z_ref[...]
    zs = z[0] + z[1]
    pe = pe_ref[...]
    pes = pe[0] + pe[1]
    s1 = pes[:, 0:1]
    cnt = jnp.maximum(pes[:, 1:2], 1.0)
    y1 = jnp.dot(x_ref[...], wt_ref[...], preferred_element_type=jnp.float32)
    s2 = jnp.dot(zs, wb_ref[...], preferred_element_type=jnp.float32)
    o_ref[...] = (s2 + y1 * s1) / cnt + b_ref[...]


def _edge_body(x_hbm, idx_hbm, pm_hbm, pe_hbm,
               idx0, idx1, idx2, rows0, rows1, ex0, ex1,
               semi0, semi1, semi2, semg0, semg1,
               semm0, semm1, seme0, seme1, semz,
               acc_m, acc_e):
    cid = lax.axis_index("c")
    sid = lax.axis_index("s")
    wid = cid * NS + sid
    idx = (idx0, idx1, idx2)
    semi = (semi0, semi1, semi2)
    rows = (rows0, rows1)
    semg = (semg0, semg1)
    ex = (ex0, ex1)
    semm = (semm0, semm1)
    seme = (seme0, seme1)

    def i_start(c, s3):
        pltpu.async_copy(idx_hbm.at[wid, c], idx[s3], semi[s3])

    def i_wait(c, s3):
        pltpu.make_async_copy(idx_hbm.at[wid, c], idx[s3], semi[s3]).wait()

    def g_start(s3, x2):
        pltpu.async_copy(x_hbm.at[idx[s3].at[1]], rows[x2], semg[x2])

    def g_wait(s3, x2):
        pltpu.make_async_copy(x_hbm.at[idx[s3].at[1]], rows[x2], semg[x2]).wait()

    lanes = lax.iota(jnp.int32, 16)

    def compute(s3, x2):
        rbuf = idx[s3]
        rows_b = rows[x2]
        ex_b = ex[x2]

        def group_body(g, _):
            eav = plsc.bitcast(rbuf[2, pl.ds(g * 16, 16)], jnp.float32)
            base = g * 16
            for e16 in range(16):
                ea = eav[e16]
                e = base + e16
                for v in range(NV):
                    rows_b[e, pl.ds(v * 16, 16)] = rows_b[e, pl.ds(v * 16, 16)] * ea
                ex_b[e, :] = jnp.where(
                    lanes == 0, ea,
                    jnp.where(lanes == 1, jnp.float32(1.0), jnp.float32(0.0)))
            return 0
        lax.fori_loop(0, CPAD // 16, group_body, 0)

    def s_start(s3, x2):
        pltpu.async_copy(rows[x2], acc_m.at[idx[s3].at[0]], semm[x2], add=True)
        pltpu.async_copy(ex[x2], acc_e.at[idx[s3].at[0]], seme[x2], add=True)

    def s_wait(s3, x2):
        pltpu.make_async_copy(rows[x2], acc_m.at[idx[s3].at[0]], semm[x2]).wait()
        pltpu.make_async_copy(ex[x2], acc_e.at[idx[s3].at[0]], seme[x2]).wait()

    def step(c, s3, x2, has_next, has_pf, has_prev):
        g_wait(s3, x2)
        if has_prev:
            s_wait((s3 + 2) % 3, 1 - x2)
        if has_next:
            i_wait(c + 1, (s3 + 1) % 3)
            g_start((s3 + 1) % 3, 1 - x2)
        compute(s3, x2)
        s_start(s3, x2)
        if has_pf:
            i_start(c + 2, (s3 + 2) % 3)

    i_start(0, 0)
    i_start(1, 1)

    def zrow(i, _):
        for v in range(NV):
            rows0[i, pl.ds(v * 16, 16)] = jnp.zeros((16,), jnp.float32)
        ex0[i, :] = jnp.zeros((16,), jnp.float32)
        return 0
    lax.fori_loop(0, CPAD, zrow, 0)
    NJ = NSTRIPE // NS
    for j in range(NJ):
        st = sid * NJ + j
        pltpu.async_copy(rows0.at[pl.ds(0, ZSTRIPE)],
                         acc_m.at[pl.ds(st * ZSTRIPE, ZSTRIPE)], semz)
        pltpu.async_copy(ex0.at[pl.ds(0, ZSTRIPE)],
                         acc_e.at[pl.ds(st * ZSTRIPE, ZSTRIPE)], semz)
    @pl.when(sid == 0)
    def _():
        pltpu.async_copy(rows0.at[pl.ds(0, ACC_N - N_NODES)],
                         acc_m.at[pl.ds(N_NODES, ACC_N - N_NODES)], semz)
        pltpu.async_copy(ex0.at[pl.ds(0, ACC_N - N_NODES)],
                         acc_e.at[pl.ds(N_NODES, ACC_N - N_NODES)], semz)
    for j in range(NJ):
        st = sid * NJ + j
        pltpu.make_async_copy(rows0.at[pl.ds(0, ZSTRIPE)],
                              acc_m.at[pl.ds(st * ZSTRIPE, ZSTRIPE)], semz).wait()
        pltpu.make_async_copy(ex0.at[pl.ds(0, ZSTRIPE)],
                              acc_e.at[pl.ds(st * ZSTRIPE, ZSTRIPE)], semz).wait()
    @pl.when(sid == 0)
    def _():
        pltpu.make_async_copy(rows0.at[pl.ds(0, ACC_N - N_NODES)],
                              acc_m.at[pl.ds(N_NODES, ACC_N - N_NODES)], semz).wait()
        pltpu.make_async_copy(ex0.at[pl.ds(0, ACC_N - N_NODES)],
                              acc_e.at[pl.ds(N_NODES, ACC_N - N_NODES)], semz).wait()
    plsc.subcore_barrier()

    i_wait(0, 0)
    g_start(0, 0)
    step(0, 0, 0, True, True, False)

    def main_body(i, _):
        c0 = i * PERIOD + 1
        for k in range(PERIOD):
            ck = k + 1
            step(c0 + k, ck % 3, ck % 2, True, True, True)
        return 0
    NMAIN = (NCHUNK - 3) // PERIOD * PERIOD
    lax.fori_loop(0, NMAIN // PERIOD, main_body, 0)

    for c in range(NMAIN + 1, NCHUNK):
        step(c, c % 3, c % 2, c + 1 < NCHUNK, c + 2 < NCHUNK, True)
    s_wait((NCHUNK - 1) % 3, (NCHUNK - 1) % 2)
    plsc.subcore_barrier()

    for j in range(NJ):
        st = sid * NJ + j
        pltpu.async_copy(acc_m.at[pl.ds(st * ZSTRIPE, ZSTRIPE)],
                         pm_hbm.at[cid, pl.ds(st * ZSTRIPE, ZSTRIPE)], semz)
        pltpu.async_copy(acc_e.at[pl.ds(st * ZSTRIPE, ZSTRIPE)],
                         pe_hbm.at[cid, pl.ds(st * ZSTRIPE, ZSTRIPE)], semz)
    for j in range(NJ):
        st = sid * NJ + j
        pltpu.make_async_copy(acc_m.at[pl.ds(st * ZSTRIPE, ZSTRIPE)],
                              pm_hbm.at[cid, pl.ds(st * ZSTRIPE, ZSTRIPE)], semz).wait()
        pltpu.make_async_copy(acc_e.at[pl.ds(st * ZSTRIPE, ZSTRIPE)],
                              pe_hbm.at[cid, pl.ds(st * ZSTRIPE, ZSTRIPE)], semz).wait()


_edge_call = pl.kernel(
    _edge_body,
    out_type=[
        jax.ShapeDtypeStruct((NC, N_NODES, IN_CH), jnp.float32),
        jax.ShapeDtypeStruct((NC, N_NODES, 16), jnp.float32),
    ],
    mesh=plsc.VectorSubcoreMesh(core_axis_name="c", subcore_axis_name="s",
                                num_cores=NC, num_subcores=NS),
    compiler_params=pltpu.CompilerParams(use_tc_tiling_on_sc=False,
                                         needs_layout_passes=False),
    scratch_types=[
        pltpu.VMEM((3, CPAD), jnp.int32),
        pltpu.VMEM((3, CPAD), jnp.int32),
        pltpu.VMEM((3, CPAD), jnp.int32),
        pltpu.VMEM((CPAD, IN_CH), jnp.float32),
        pltpu.VMEM((CPAD, IN_CH), jnp.float32),
        pltpu.VMEM((CPAD, 16), jnp.float32),
        pltpu.VMEM((CPAD, 16), jnp.float32),
        pltpu.SemaphoreType.DMA,
        pltpu.SemaphoreType.DMA,
        pltpu.SemaphoreType.DMA,
        pltpu.SemaphoreType.DMA,
        pltpu.SemaphoreType.DMA,
        pltpu.SemaphoreType.DMA,
        pltpu.SemaphoreType.DMA,
        pltpu.SemaphoreType.DMA,
        pltpu.SemaphoreType.DMA,
        pltpu.SemaphoreType.DMA,
        pltpu.VMEM_SHARED((ACC_N, IN_CH), jnp.float32),
        pltpu.VMEM_SHARED((ACC_N, 16), jnp.float32),
    ],
)

FBLK = 2000

_final_call = pl.pallas_call(
    _final_body,
    grid=(N_NODES // FBLK,),
    in_specs=[
        pl.BlockSpec((FBLK, IN_CH), lambda i: (i, 0)),
        pl.BlockSpec((NC, FBLK, IN_CH), lambda i: (0, i, 0)),
        pl.BlockSpec((NC, FBLK, 16), lambda i: (0, i, 0)),
        pl.BlockSpec((IN_CH, OUT_CH), lambda i: (0, 0)),
        pl.BlockSpec((IN_CH, OUT_CH), lambda i: (0, 0)),
        pl.BlockSpec((1, OUT_CH), lambda i: (0, 0)),
    ],
    out_specs=pl.BlockSpec((FBLK, OUT_CH), lambda i: (i, 0)),
    out_shape=jax.ShapeDtypeStruct((N_NODES, OUT_CH), jnp.float32),
)


@jax.jit
def kernel(x, edge_index, edge_attr, weight, bias):
    ei = edge_index.astype(jnp.int32).reshape(2, NW, NCHUNK, CHUNK)
    eab = lax.bitcast_convert_type(edge_attr, jnp.int32).reshape(NW, NCHUNK, CHUNK)
    rowp = jnp.full((NW, NCHUNK, CPAD - CHUNK), N_NODES, jnp.int32)
    zp = jnp.zeros((NW, NCHUNK, CPAD - CHUNK), jnp.int32)
    idx3 = jnp.stack([
        jnp.concatenate([ei[0], rowp], axis=2),
        jnp.concatenate([ei[1], zp], axis=2),
        jnp.concatenate([eab, zp], axis=2),
    ], axis=2)
    zm, pe = _edge_call(x, idx3)
    return _final_call(x, zm, pe, weight[:IN_CH], weight[IN_CH:],
                       bias.reshape(1, OUT_CH))
